# baseline (speedup 1.0000x reference)
import jax
import jax.numpy as jnp
from jax import lax
from jax.experimental import pallas as pl
from jax.experimental.pallas import tpu as pltpu
from jax.experimental.pallas import tpu_sc as plsc

N = 10000
E = 320000
C = 128

NC = 2
NS = 16
NW = NC * NS
EPT = 9984
REM = E - NW * EPT
SUP = 768
NSUP = EPT // SUP
CHUNK = 64
NCHUNK = SUP // CHUNK
STRIPE = 624
TAIL = N - NS * STRIPE
ZROWS = 16
ZD = 1024
NPAD = 10240

BR = 1000


def _pre_body(x_ref, w_ref, h_ref):
    h_ref[...] = jnp.dot(x_ref[...], w_ref[...],
                         preferred_element_type=jnp.float32)


_pre = pl.pallas_call(
    _pre_body,
    grid=(N // BR,),
    in_specs=[
        pl.BlockSpec((BR, C), lambda i: (i, 0)),
        pl.BlockSpec((C, C), lambda i: (0, 0)),
    ],
    out_specs=pl.BlockSpec((BR, C), lambda i: (i, 0)),
    out_shape=jax.ShapeDtypeStruct((N, C), jnp.float32),
)


def _att_body(h_ref, s_ref, d_ref, aa_ref):
    h = h_ref[...]
    asv = jnp.sum(h * s_ref[...], axis=1)
    adv = jnp.sum(h * d_ref[...], axis=1)
    aa_ref[...] = jnp.stack([asv, adv], axis=0)


_att = pl.pallas_call(
    _att_body,
    in_specs=[
        pl.BlockSpec((N, C), lambda: (0, 0)),
        pl.BlockSpec((1, C), lambda: (0, 0)),
        pl.BlockSpec((1, C), lambda: (0, 0)),
    ],
    out_specs=pl.BlockSpec((2, N), lambda: (0, 0)),
    out_shape=jax.ShapeDtypeStruct((2, N), jnp.float32),
)


def _make_gat_sc():
    mesh = plsc.VectorSubcoreMesh(core_axis_name="c", subcore_axis_name="s")

    def body(h_hbm, aa_hbm, ei_hbm,
             u_hbm, den_hbm,
             aa_v, ed_v, e_c0, dst_c0, rows_v0, e_c1, dst_c1, rows_v1,
             e_c2, dst_c2, rows_v2,
             zbuf, zden, out_sh, den_sh,
             gsem0, gsem1, gsem2, ssem0, ssem1, ssem2):
        cid = lax.axis_index("c")
        sid = lax.axis_index("s")
        wid = cid * NS + sid

        zero16 = jnp.zeros((16,), jnp.float32)
        zero16i = jnp.zeros((16,), jnp.int32)
        row_s = zero16i
        row_d = zero16i + 1

        pltpu.sync_copy(aa_hbm, aa_v)

        def _zd(i, carry):
            zden[pl.ds(i * 16, 16)] = zero16
            return carry
        lax.fori_loop(0, zden.shape[0] // 16, _zd, 0)
        for r in range(ZROWS):
            for q in range(C // 16):
                zbuf[r, pl.ds(q * 16, 16)] = zero16

        for k in range(STRIPE // ZROWS):
            pltpu.sync_copy(zbuf, out_sh.at[pl.ds(sid * STRIPE + k * ZROWS, ZROWS), :])

        @pl.when(sid == 0)
        def _zero_tail():
            pltpu.sync_copy(zbuf, out_sh.at[pl.ds(NS * STRIPE, TAIL), :])
            for k in range(NPAD // ZD):
                pltpu.sync_copy(zden, den_sh.at[pl.ds(k * ZD, ZD)])

        plsc.subcore_barrier()

        bufs = ((rows_v0, e_c0, dst_c0, gsem0, ssem0),
                (rows_v1, e_c1, dst_c1, gsem1, ssem1),
                (rows_v2, e_c2, dst_c2, gsem2, ssem2))

        def e_compute(base, b, n=CHUNK):
            _, e_c, dst_c, _, _ = bufs[b]
            for s in range(n // 16):
                off = base + s * 16
                si = ed_v[0, pl.ds(off, 16)]
                di = ed_v[1, pl.ds(off, 16)]
                av = plsc.load_gather(aa_v, [row_s, si])
                bv = plsc.load_gather(aa_v, [row_d, di])
                al = av + bv
                al = jnp.where(al >= 0.0, al, al * jnp.float32(0.2))
                ev = jnp.exp(al)
                e_c[pl.ds(s * 16, 16)] = ev
                dst_c[pl.ds(s * 16, 16)] = di

        def scale(b, n=CHUNK):
            rows_v, e_c, _, _, _ = bufs[b]

            def scale_body(r2, carry2):
                for u in range(4):
                    r = r2 * 4 + u
                    bc = plsc.load_gather(e_c, [jnp.full((16,), r, jnp.int32)])
                    for q in range(C // 16):
                        sl = pl.ds(q * 16, 16)
                        rows_v[r, sl] = rows_v[r, sl] * bc
                return carry2
            lax.fori_loop(0, n // 4, scale_body, 0)

        def issue_gather(base, b, n=CHUNK):
            rows_v, _, _, gsem, _ = bufs[b]
            if n == CHUNK:
                dst = rows_v
            else:
                dst = rows_v.at[pl.ds(0, n), :]
            pltpu.async_copy(h_hbm.at[ed_v.at[0, pl.ds(base, n)]], dst, gsem)

        def wait_gather(base, b, n=CHUNK):
            rows_v, _, _, gsem, _ = bufs[b]
            if n == CHUNK:
                dst = rows_v
            else:
                dst = rows_v.at[pl.ds(0, n), :]
            pltpu.make_async_copy(h_hbm.at[ed_v.at[0, pl.ds(base, n)]],
                                  dst, gsem).wait()

        def issue_scatter(b, n=CHUNK):
            rows_v, e_c, dst_c, _, ssem = bufs[b]
            if n == CHUNK:
                rsrc, esrc, idx = rows_v, e_c, dst_c
            else:
                rsrc = rows_v.at[pl.ds(0, n), :]
                esrc = e_c.at[pl.ds(0, n)]
                idx = dst_c.at[pl.ds(0, n)]
            pltpu.async_copy(rsrc, out_sh.at[idx], ssem, add=True)
            pltpu.async_copy(esrc, den_sh.at[idx], ssem, add=True)

        def wait_scatter(b, n=CHUNK):
            rows_v, e_c, dst_c, _, ssem = bufs[b]
            if n == CHUNK:
                rsrc, esrc, idx = rows_v, e_c, dst_c
            else:
                rsrc = rows_v.at[pl.ds(0, n), :]
                esrc = e_c.at[pl.ds(0, n)]
                idx = dst_c.at[pl.ds(0, n)]
            pltpu.make_async_copy(rsrc, out_sh.at[idx], ssem).wait()
            pltpu.make_async_copy(esrc, den_sh.at[idx], ssem).wait()

        for b in range(3):
            rows_v, e_c, dst_c, _, _ = bufs[b]

            def _zr(r, carry, _rv=rows_v):
                ri = jnp.full((16,), r, jnp.int32)
                for q in range(C // 16):
                    plsc.store_scatter(_rv, [ri, lax.iota(jnp.int32, 16) + q * 16],
                                       zero16)
                return carry
            lax.fori_loop(0, CHUNK, _zr, 0)
            for s in range(CHUNK // 16):
                e_c[pl.ds(s * 16, 16)] = zero16
                dst_c[pl.ds(s * 16, 16)] = zero16i
            issue_scatter(b)

        def step(c, b, bn, last):
            if not last:
                wait_scatter(bn)
                e_compute(c + CHUNK, bn)
                issue_gather(c + CHUNK, bn)
            wait_gather(c, b)
            scale(b)
            issue_scatter(b)

        def sup_body(si_, carry0):
            ebase = wid * EPT + si_ * SUP
            pltpu.sync_copy(ei_hbm.at[:, pl.ds(ebase, SUP)], ed_v)

            wait_scatter(0)
            e_compute(0, 0)
            issue_gather(0, 0)

            def triple_body(j, carry):
                c = 3 * j * CHUNK
                step(c, 0, 1, False)
                step(c + CHUNK, 1, 2, False)

                @pl.when(j < NCHUNK // 3 - 1)
                def _mid():
                    step(c + 2 * CHUNK, 2, 0, False)

                @pl.when(j == NCHUNK // 3 - 1)
                def _last():
                    step(c + 2 * CHUNK, 2, 0, True)
                return carry
            lax.fori_loop(0, NCHUNK // 3, triple_body, 0)
            return carry0
        lax.fori_loop(0, NSUP, sup_body, 0)

        @pl.when(wid < 4)
        def _rem():
            rb = NW * EPT + wid * (REM // 4)
            pltpu.sync_copy(ei_hbm.at[:, pl.ds(rb, REM // 4)],
                            ed_v.at[:, pl.ds(0, REM // 4)])
            for t in range(2):
                wait_scatter(0)
                e_compute(t * CHUNK, 0)
                issue_gather(t * CHUNK, 0)
                wait_gather(t * CHUNK, 0)
                scale(0)
                issue_scatter(0)

        wait_scatter(0)
        wait_scatter(1)
        wait_scatter(2)

        plsc.subcore_barrier()

        pltpu.sync_copy(out_sh.at[pl.ds(sid * STRIPE, STRIPE), :],
                        u_hbm.at[pl.ds(cid * N + sid * STRIPE, STRIPE), :])

        @pl.when(sid == 0)
        def _write_tail():
            pltpu.sync_copy(out_sh.at[pl.ds(NS * STRIPE, TAIL), :],
                            u_hbm.at[pl.ds(cid * N + NS * STRIPE, TAIL), :])
            for k in range(NPAD // ZD):
                pltpu.sync_copy(den_sh.at[pl.ds(k * ZD, ZD)], zden)
                pltpu.sync_copy(zden, den_hbm.at[cid, pl.ds(k * ZD, ZD)])

    return pl.kernel(
        body,
        out_type=[
            jax.ShapeDtypeStruct((NC * N, C), jnp.float32),
            jax.ShapeDtypeStruct((NC, NPAD), jnp.float32),
        ],
        mesh=mesh,
        compiler_params=pltpu.CompilerParams(needs_layout_passes=False),
        scratch_types=[
            pltpu.VMEM((2, N), jnp.float32),
            pltpu.VMEM((2, SUP), jnp.int32),
            pltpu.VMEM((CHUNK,), jnp.float32),
            pltpu.VMEM((CHUNK,), jnp.int32),
            pltpu.VMEM((CHUNK, C), jnp.float32),
            pltpu.VMEM((CHUNK,), jnp.float32),
            pltpu.VMEM((CHUNK,), jnp.int32),
            pltpu.VMEM((CHUNK, C), jnp.float32),
            pltpu.VMEM((CHUNK,), jnp.float32),
            pltpu.VMEM((CHUNK,), jnp.int32),
            pltpu.VMEM((CHUNK, C), jnp.float32),
            pltpu.VMEM((ZROWS, C), jnp.float32),
            pltpu.VMEM((ZD,), jnp.float32),
            pltpu.VMEM_SHARED((N, C), jnp.float32),
            pltpu.VMEM_SHARED((NPAD,), jnp.float32),
            pltpu.SemaphoreType.DMA,
            pltpu.SemaphoreType.DMA,
            pltpu.SemaphoreType.DMA,
            pltpu.SemaphoreType.DMA,
            pltpu.SemaphoreType.DMA,
            pltpu.SemaphoreType.DMA,
        ],
    )


_gat_sc = _make_gat_sc()


def _post_body(u0_ref, u1_ref, den_ref, w2_ref, wl1_ref, b1_ref,
               wl2_ref, b2_ref, h2_ref, h4_ref):
    den = (den_ref[:, 0:1] + den_ref[:, 1:2] + jnp.float32(1e-16))
    h1 = (u0_ref[...] + u1_ref[...]) / den
    h1 = jnp.where(h1 > 0.0, h1, jnp.exp(h1) - 1.0)
    h2 = jnp.dot(h1.astype(jnp.bfloat16), w2_ref[...].astype(jnp.bfloat16),
                 preferred_element_type=jnp.float32)
    h2_ref[...] = h2
    h3 = lax.dot_general(h2.astype(jnp.bfloat16),
                         wl1_ref[...].astype(jnp.bfloat16),
                         (((1,), (1,)), ((), ())),
                         preferred_element_type=jnp.float32) + b1_ref[...]
    h3 = jnp.where(h3 > 0.0, h3, jnp.exp(h3) - 1.0)
    h4_ref[...] = lax.dot_general(h3.astype(jnp.bfloat16),
                                  wl2_ref[...].astype(jnp.bfloat16),
                                  (((1,), (1,)), ((), ())),
                                  preferred_element_type=jnp.float32) + b2_ref[...]


_post = pl.pallas_call(
    _post_body,
    grid=(N // BR,),
    in_specs=[
        pl.BlockSpec((BR, C), lambda i: (i, 0)),
        pl.BlockSpec((BR, C), lambda i: (i + N // BR, 0)),
        pl.BlockSpec((BR, 2), lambda i: (i, 0)),
        pl.BlockSpec((C, C), lambda i: (0, 0)),
        pl.BlockSpec((C, C), lambda i: (0, 0)),
        pl.BlockSpec((1, C), lambda i: (0, 0)),
        pl.BlockSpec((C, C), lambda i: (0, 0)),
        pl.BlockSpec((1, C), lambda i: (0, 0)),
    ],
    out_specs=[
        pl.BlockSpec((BR, C), lambda i: (i, 0)),
        pl.BlockSpec((BR, C), lambda i: (i, 0)),
    ],
    out_shape=[
        jax.ShapeDtypeStruct((N, C), jnp.float32),
        jax.ShapeDtypeStruct((N, C), jnp.float32),
    ],
)


def kernel(features, edge_index, W1, att_src1, att_dst1, W2, Wl1, b1, Wl2, b2):
    h = _pre(features, W1)
    aa = _att(h, att_src1.reshape(1, C), att_dst1.reshape(1, C))
    u, den = _gat_sc(h, aa, edge_index)
    den_t = den[:, :N].T
    h2, h4 = _post(u, u, den_t, W2, Wl1, b1.reshape(1, C), Wl2, b2.reshape(1, C))
    return (h2, h4)


# speedup vs baseline: 1.0038x; 1.0038x over previous
import jax
import jax.numpy as jnp
from jax import lax
from jax.experimental import pallas as pl
from jax.experimental.pallas import tpu as pltpu
from jax.experimental.pallas import tpu_sc as plsc

N = 10000
E = 320000
C = 128

NC = 2
NS = 16
NW = NC * NS
EPT = 9984
REM = E - NW * EPT
SUP = 768
NSUP = EPT // SUP
CHUNK = 64
NCHUNK = SUP // CHUNK
STRIPE = 624
TAIL = N - NS * STRIPE
ZROWS = 16
ZD = 1024
NPAD = 10240

BR = 1000


def _pre_body(x_ref, w_ref, h_ref):
    h_ref[...] = jnp.dot(x_ref[...], w_ref[...],
                         preferred_element_type=jnp.float32)


_pre = pl.pallas_call(
    _pre_body,
    grid=(N // BR,),
    in_specs=[
        pl.BlockSpec((BR, C), lambda i: (i, 0)),
        pl.BlockSpec((C, C), lambda i: (0, 0)),
    ],
    out_specs=pl.BlockSpec((BR, C), lambda i: (i, 0)),
    out_shape=jax.ShapeDtypeStruct((N, C), jnp.float32),
)


def _att_body(h_ref, s_ref, d_ref, aa_ref):
    h = h_ref[...]
    asv = jnp.sum(h * s_ref[...], axis=1)
    adv = jnp.sum(h * d_ref[...], axis=1)
    aa_ref[...] = jnp.stack([asv, adv], axis=0)


_att = pl.pallas_call(
    _att_body,
    in_specs=[
        pl.BlockSpec((N, C), lambda: (0, 0)),
        pl.BlockSpec((1, C), lambda: (0, 0)),
        pl.BlockSpec((1, C), lambda: (0, 0)),
    ],
    out_specs=pl.BlockSpec((2, N), lambda: (0, 0)),
    out_shape=jax.ShapeDtypeStruct((2, N), jnp.float32),
)


def _make_gat_sc():
    mesh = plsc.VectorSubcoreMesh(core_axis_name="c", subcore_axis_name="s")

    def body(h_hbm, aa_hbm, ei_hbm,
             u_hbm, den_hbm,
             aa_v, ed_v, e_c0, dst_c0, rows_v0, e_c1, dst_c1, rows_v1,
             e_c2, dst_c2, rows_v2,
             zbuf, zden, out_sh, den_sh,
             gsem0, gsem1, gsem2, ssem0, ssem1, ssem2):
        cid = lax.axis_index("c")
        sid = lax.axis_index("s")
        wid = cid * NS + sid

        zero16 = jnp.zeros((16,), jnp.float32)
        zero16i = jnp.zeros((16,), jnp.int32)
        row_s = zero16i
        row_d = zero16i + 1

        pltpu.sync_copy(aa_hbm, aa_v)

        bufs = ((rows_v0, e_c0, dst_c0, gsem0, ssem0),
                (rows_v1, e_c1, dst_c1, gsem1, ssem1),
                (rows_v2, e_c2, dst_c2, gsem2, ssem2))

        def _zd(i, carry):
            zden[pl.ds(i * 16, 16)] = zero16
            return carry
        lax.fori_loop(0, zden.shape[0] // 16, _zd, 0)
        for r in range(ZROWS):
            for q in range(C // 16):
                zbuf[r, pl.ds(q * 16, 16)] = zero16

        for k in range(STRIPE // ZROWS):
            pltpu.sync_copy(zbuf, out_sh.at[pl.ds(sid * STRIPE + k * ZROWS, ZROWS), :])

        @pl.when(sid == 0)
        def _zero_tail():
            pltpu.sync_copy(zbuf, out_sh.at[pl.ds(NS * STRIPE, TAIL), :])
            for k in range(NPAD // ZD):
                pltpu.sync_copy(zden, den_sh.at[pl.ds(k * ZD, ZD)])

        plsc.subcore_barrier()

        def e_compute(ed_v, base, b, n=CHUNK):
            _, e_c, dst_c, _, _ = bufs[b]
            for s in range(n // 16):
                off = base + s * 16
                si = ed_v[0, pl.ds(off, 16)]
                di = ed_v[1, pl.ds(off, 16)]
                av = plsc.load_gather(aa_v, [row_s, si])
                bv = plsc.load_gather(aa_v, [row_d, di])
                al = av + bv
                al = jnp.where(al >= 0.0, al, al * jnp.float32(0.2))
                ev = jnp.exp(al)
                e_c[pl.ds(s * 16, 16)] = ev
                dst_c[pl.ds(s * 16, 16)] = di

        def scale(b, n=CHUNK):
            rows_v, e_c, _, _, _ = bufs[b]

            def scale_body(r2, carry2):
                for u in range(4):
                    r = r2 * 4 + u
                    bc = plsc.load_gather(e_c, [jnp.full((16,), r, jnp.int32)])
                    for q in range(C // 16):
                        sl = pl.ds(q * 16, 16)
                        rows_v[r, sl] = rows_v[r, sl] * bc
                return carry2
            lax.fori_loop(0, n // 4, scale_body, 0)

        def issue_gather(ed_v, base, b, n=CHUNK):
            rows_v, _, _, gsem, _ = bufs[b]
            if n == CHUNK:
                dst = rows_v
            else:
                dst = rows_v.at[pl.ds(0, n), :]
            pltpu.async_copy(h_hbm.at[ed_v.at[0, pl.ds(base, n)]], dst, gsem)

        def wait_gather(ed_v, base, b, n=CHUNK):
            rows_v, _, _, gsem, _ = bufs[b]
            if n == CHUNK:
                dst = rows_v
            else:
                dst = rows_v.at[pl.ds(0, n), :]
            pltpu.make_async_copy(h_hbm.at[ed_v.at[0, pl.ds(base, n)]],
                                  dst, gsem).wait()

        def issue_scatter(b, n=CHUNK):
            rows_v, e_c, dst_c, _, ssem = bufs[b]
            if n == CHUNK:
                rsrc, esrc, idx = rows_v, e_c, dst_c
            else:
                rsrc = rows_v.at[pl.ds(0, n), :]
                esrc = e_c.at[pl.ds(0, n)]
                idx = dst_c.at[pl.ds(0, n)]
            pltpu.async_copy(rsrc, out_sh.at[idx], ssem, add=True)
            pltpu.async_copy(esrc, den_sh.at[idx], ssem, add=True)

        def wait_scatter(b, n=CHUNK):
            rows_v, e_c, dst_c, _, ssem = bufs[b]
            if n == CHUNK:
                rsrc, esrc, idx = rows_v, e_c, dst_c
            else:
                rsrc = rows_v.at[pl.ds(0, n), :]
                esrc = e_c.at[pl.ds(0, n)]
                idx = dst_c.at[pl.ds(0, n)]
            pltpu.make_async_copy(rsrc, out_sh.at[idx], ssem).wait()
            pltpu.make_async_copy(esrc, den_sh.at[idx], ssem).wait()

        for b in range(3):
            rows_v, e_c, dst_c, _, _ = bufs[b]

            def _zr(r, carry, _rv=rows_v):
                ri = jnp.full((16,), r, jnp.int32)
                for q in range(C // 16):
                    plsc.store_scatter(_rv, [ri, lax.iota(jnp.int32, 16) + q * 16],
                                       zero16)
                return carry
            lax.fori_loop(0, CHUNK, _zr, 0)
            for s in range(CHUNK // 16):
                e_c[pl.ds(s * 16, 16)] = zero16
                dst_c[pl.ds(s * 16, 16)] = zero16i
            issue_scatter(b)

        def step(c, b, bn, last):
            if not last:
                wait_scatter(bn)
                e_compute(ed_v, c + CHUNK, bn)
                issue_gather(ed_v, c + CHUNK, bn)
            wait_gather(ed_v, c, b)
            scale(b)
            issue_scatter(b)

        def sup_body(si_, carry0):
            ebase = wid * EPT + si_ * SUP
            pltpu.sync_copy(ei_hbm.at[:, pl.ds(ebase, SUP)], ed_v)

            wait_scatter(0)
            e_compute(ed_v, 0, 0)
            issue_gather(ed_v, 0, 0)

            def triple_body(j, carry):
                c = 3 * j * CHUNK
                step(c, 0, 1, False)
                step(c + CHUNK, 1, 2, False)

                @pl.when(j < NCHUNK // 3 - 1)
                def _mid():
                    step(c + 2 * CHUNK, 2, 0, False)

                @pl.when(j == NCHUNK // 3 - 1)
                def _last():
                    step(c + 2 * CHUNK, 2, 0, True)
                return carry
            lax.fori_loop(0, NCHUNK // 3, triple_body, 0)
            return carry0
        lax.fori_loop(0, NSUP, sup_body, 0)

        @pl.when(wid < 4)
        def _rem():
            rb = NW * EPT + wid * (REM // 4)
            pltpu.sync_copy(ei_hbm.at[:, pl.ds(rb, REM // 4)],
                            ed_v.at[:, pl.ds(0, REM // 4)])
            for t in range(2):
                wait_scatter(0)
                e_compute(ed_v, t * CHUNK, 0)
                issue_gather(ed_v, t * CHUNK, 0)
                wait_gather(ed_v, t * CHUNK, 0)
                scale(0)
                issue_scatter(0)

        wait_scatter(0)
        wait_scatter(1)
        wait_scatter(2)

        plsc.subcore_barrier()

        pltpu.sync_copy(out_sh.at[pl.ds(sid * STRIPE, STRIPE), :],
                        u_hbm.at[pl.ds(cid * N + sid * STRIPE, STRIPE), :])

        @pl.when(sid == 0)
        def _write_tail():
            pltpu.sync_copy(out_sh.at[pl.ds(NS * STRIPE, TAIL), :],
                            u_hbm.at[pl.ds(cid * N + NS * STRIPE, TAIL), :])
            for k in range(NPAD // ZD):
                pltpu.sync_copy(den_sh.at[pl.ds(k * ZD, ZD)], zden)
                pltpu.sync_copy(zden, den_hbm.at[cid, pl.ds(k * ZD, ZD)])

    return pl.kernel(
        body,
        out_type=[
            jax.ShapeDtypeStruct((NC * N, C), jnp.float32),
            jax.ShapeDtypeStruct((NC, NPAD), jnp.float32),
        ],
        mesh=mesh,
        compiler_params=pltpu.CompilerParams(needs_layout_passes=False),
        scratch_types=[
            pltpu.VMEM((2, N), jnp.float32),
            pltpu.VMEM((2, SUP), jnp.int32),
            pltpu.VMEM((CHUNK,), jnp.float32),
            pltpu.VMEM((CHUNK,), jnp.int32),
            pltpu.VMEM((CHUNK, C), jnp.float32),
            pltpu.VMEM((CHUNK,), jnp.float32),
            pltpu.VMEM((CHUNK,), jnp.int32),
            pltpu.VMEM((CHUNK, C), jnp.float32),
            pltpu.VMEM((CHUNK,), jnp.float32),
            pltpu.VMEM((CHUNK,), jnp.int32),
            pltpu.VMEM((CHUNK, C), jnp.float32),
            pltpu.VMEM((ZROWS, C), jnp.float32),
            pltpu.VMEM((ZD,), jnp.float32),
            pltpu.VMEM_SHARED((N, C), jnp.float32),
            pltpu.VMEM_SHARED((NPAD,), jnp.float32),
            pltpu.SemaphoreType.DMA,
            pltpu.SemaphoreType.DMA,
            pltpu.SemaphoreType.DMA,
            pltpu.SemaphoreType.DMA,
            pltpu.SemaphoreType.DMA,
            pltpu.SemaphoreType.DMA,
        ],
    )


_gat_sc = _make_gat_sc()


def _post_body(u0_ref, u1_ref, den_ref, w2_ref, wl1_ref, b1_ref,
               wl2_ref, b2_ref, h2_ref, h4_ref):
    den = (den_ref[:, 0:1] + den_ref[:, 1:2] + jnp.float32(1e-16))
    h1 = (u0_ref[...] + u1_ref[...]) / den
    h1 = jnp.where(h1 > 0.0, h1, jnp.exp(h1) - 1.0)
    h2 = jnp.dot(h1, w2_ref[...], preferred_element_type=jnp.float32)
    h2_ref[...] = h2
    h3 = lax.dot_general(h2, wl1_ref[...], (((1,), (1,)), ((), ())),
                         preferred_element_type=jnp.float32) + b1_ref[...]
    h3 = jnp.where(h3 > 0.0, h3, jnp.exp(h3) - 1.0)
    h4_ref[...] = lax.dot_general(h3, wl2_ref[...], (((1,), (1,)), ((), ())),
                                  preferred_element_type=jnp.float32) + b2_ref[...]


_post = pl.pallas_call(
    _post_body,
    grid=(N // BR,),
    in_specs=[
        pl.BlockSpec((BR, C), lambda i: (i, 0)),
        pl.BlockSpec((BR, C), lambda i: (i + N // BR, 0)),
        pl.BlockSpec((BR, 2), lambda i: (i, 0)),
        pl.BlockSpec((C, C), lambda i: (0, 0)),
        pl.BlockSpec((C, C), lambda i: (0, 0)),
        pl.BlockSpec((1, C), lambda i: (0, 0)),
        pl.BlockSpec((C, C), lambda i: (0, 0)),
        pl.BlockSpec((1, C), lambda i: (0, 0)),
    ],
    out_specs=[
        pl.BlockSpec((BR, C), lambda i: (i, 0)),
        pl.BlockSpec((BR, C), lambda i: (i, 0)),
    ],
    out_shape=[
        jax.ShapeDtypeStruct((N, C), jnp.float32),
        jax.ShapeDtypeStruct((N, C), jnp.float32),
    ],
)


def kernel(features, edge_index, W1, att_src1, att_dst1, W2, Wl1, b1, Wl2, b2):
    h = _pre(features, W1)
    aa = _att(h, att_src1.reshape(1, C), att_dst1.reshape(1, C))
    u, den = _gat_sc(h, aa, edge_index)
    den_t = den[:, :N].T
    h2, h4 = _post(u, u, den_t, W2, Wl1, b1.reshape(1, C), Wl2, b2.reshape(1, C))
    return (h2, h4)
